# baseline (speedup 1.0000x reference)
import functools

import jax
import jax.numpy as jnp
from jax import lax
from jax.experimental import pallas as pl
from jax.experimental.pallas import tpu as pltpu
from jax.experimental.pallas import tpu_sc as plsc

N = 10000
E = 160000
IN_DIM = 32
OUT_DIM = 32
E_DIM = 6
EDGE_H = 256
EPS = 1e-5

ROW = 48
BLK = 640
NB = E // BLK

NC = 2
NS = 16
NW = NC * NS
EPW = E // NW
CW = 128
CH = 39
TAIL = EPW - CH * CW
LAG = 3
NBUF = 6
NGRP = 6
NTI = 10
NPT = N // NTI


def _sc_gather(h, src3, srct, *, e2, ch, tail, ngrp):
    epw = ch * CW + tail
    mesh = plsc.VectorSubcoreMesh(core_axis_name="c", subcore_axis_name="s")

    @functools.partial(
        pl.kernel,
        out_type=jax.ShapeDtypeStruct((e2, IN_DIM), jnp.float32),
        mesh=mesh,
        scratch_types=[
            pltpu.VMEM((ch, CW), jnp.int32),
            pltpu.VMEM((1, tail), jnp.int32),
            [pltpu.VMEM((CW, IN_DIM), jnp.float32)] * NBUF,
            [pltpu.SemaphoreType.DMA] * NBUF,
            [pltpu.SemaphoreType.DMA] * NBUF,
        ],
        compiler_params=pltpu.CompilerParams(use_tc_tiling_on_sc=False),
    )
    def k(h_hbm, src_hbm, srct_hbm, out_hbm, idx_v, idxt_v, rows, gsem, wsem):
        wid = lax.axis_index("s") * NC + lax.axis_index("c")
        base = wid * epw

        def start_gather(j, b):
            pltpu.async_copy(h_hbm.at[idx_v.at[j]], rows[b], gsem[b])

        def wait_gather(j, b):
            pltpu.make_async_copy(
                h_hbm.at[idx_v.at[j]], rows[b], gsem[b]).wait()

        def start_write(j, b):
            pltpu.async_copy(
                rows[b], out_hbm.at[pl.ds(base + j * CW, CW)], wsem[b])

        def wait_write(j, b):
            pltpu.make_async_copy(
                rows[b], out_hbm.at[pl.ds(base + j * CW, CW)], wsem[b]).wait()

        pltpu.sync_copy(src_hbm.at[wid], idx_v)
        pltpu.sync_copy(srct_hbm.at[wid], idxt_v)
        for d in range(LAG):
            start_gather(d, d)

        def body(g, carry):
            for d in range(NBUF):
                j = g * NBUF + d
                bn = (d + LAG) % NBUF

                @pl.when(j >= LAG)
                def _():
                    wait_write(j - LAG, bn)

                @pl.when(j + LAG < ch)
                def _():
                    start_gather(j + LAG, bn)

                wait_gather(j, d)
                start_write(j, d)
            return carry

        lax.fori_loop(0, ngrp, body, 0)
        for j in range(ngrp * NBUF, ch):
            wait_write(j - LAG, (j - LAG) % NBUF)
            wait_gather(j, j % NBUF)
            start_write(j, j % NBUF)
        for j in range(ch - LAG, ch):
            wait_write(j, j % NBUF)
        pltpu.async_copy(h_hbm.at[idxt_v.at[0]],
                         rows[LAG].at[pl.ds(0, tail)], gsem[LAG]).wait()
        pltpu.sync_copy(rows[LAG].at[pl.ds(0, tail)],
                        out_hbm.at[pl.ds(base + ch * CW, tail)])

    return k(h, src3, srct)


def _sc_scatter(msg48, dst3, dstt, prime, *, ch, tail, ngrp):
    epw = ch * CW + tail
    mesh = plsc.VectorSubcoreMesh(core_axis_name="c", subcore_axis_name="s")

    @functools.partial(
        pl.kernel,
        out_type=jax.ShapeDtypeStruct((NC * N, ROW), jnp.float32),
        mesh=mesh,
        scratch_types=[
            pltpu.VMEM((ch, CW), jnp.int32),
            pltpu.VMEM((1, tail), jnp.int32),
            [pltpu.VMEM((CW, ROW), jnp.float32)] * NBUF,
            [pltpu.SemaphoreType.DMA] * NBUF,
            [pltpu.SemaphoreType.DMA] * NBUF,
            pltpu.SemaphoreType.DMA,
            pltpu.VMEM_SHARED((N, ROW), jnp.float32),
        ],
        compiler_params=pltpu.CompilerParams(use_tc_tiling_on_sc=False),
    )
    def k(msg_hbm, dst_hbm, dstt_hbm, zero_hbm, out_hbm, idx_v, idxt_v, vals,
          rsem, ssem, zsem, acc):
        c = lax.axis_index("c")
        s = lax.axis_index("s")
        wid = s * NC + c
        base = wid * epw

        def start_read(j, b):
            pltpu.async_copy(
                msg_hbm.at[pl.ds(base + j * CW, CW)], vals[b], rsem[b])

        def wait_read(j, b):
            pltpu.make_async_copy(
                msg_hbm.at[pl.ds(base + j * CW, CW)], vals[b], rsem[b]).wait()

        def start_scat(j, b):
            pltpu.async_copy(vals[b], acc.at[idx_v.at[j]], ssem[b], add=True)

        def wait_scat(j, b):
            pltpu.make_async_copy(
                vals[b], acc.at[idx_v.at[j]], ssem[b]).wait()

        @pl.when(s < NTI)
        def _():
            pltpu.async_copy(zero_hbm.at[pl.ds(c * N + s * NPT, NPT)],
                             acc.at[pl.ds(s * NPT, NPT)], zsem).wait()

        pltpu.sync_copy(dst_hbm.at[wid], idx_v)
        pltpu.sync_copy(dstt_hbm.at[wid], idxt_v)
        plsc.subcore_barrier()
        for d in range(LAG):
            start_read(d, d)

        def body(g, carry):
            for d in range(NBUF):
                j = g * NBUF + d
                bn = (d + LAG) % NBUF

                @pl.when(j >= LAG)
                def _():
                    wait_scat(j - LAG, bn)

                @pl.when(j + LAG < ch)
                def _():
                    start_read(j + LAG, bn)

                wait_read(j, d)
                start_scat(j, d)
            return carry

        lax.fori_loop(0, ngrp, body, 0)
        for j in range(ngrp * NBUF, ch):
            wait_scat(j - LAG, (j - LAG) % NBUF)
            wait_read(j, j % NBUF)
            start_scat(j, j % NBUF)
        for j in range(ch - LAG, ch):
            wait_scat(j, j % NBUF)
        pltpu.sync_copy(msg_hbm.at[pl.ds(base + ch * CW, tail)],
                        vals[LAG].at[pl.ds(0, tail)])
        pltpu.sync_copy(vals[LAG].at[pl.ds(0, tail)],
                        acc.at[idxt_v.at[0]], add=True)
        plsc.subcore_barrier()

        @pl.when(s < NTI)
        def _():
            pltpu.sync_copy(acc.at[pl.ds(s * NPT, NPT)],
                            out_hbm.at[pl.ds(c * N + s * NPT, NPT)])

    return k(msg48, dst3, dstt, prime)


def _tc_edge_body(eT_ref, hs_ref, w1T_ref, b1T_ref, w2T_ref, b2T_ref, o_ref):
    hidT = jnp.dot(w1T_ref[...], eT_ref[...],
                   preferred_element_type=jnp.float32)
    hidT = jnp.maximum(hidT + b1T_ref[...], 0.0).astype(jnp.bfloat16)
    pT = jnp.dot(w2T_ref[...], hidT, preferred_element_type=jnp.float32)
    pT = pT + b2T_ref[...]
    hsT = hs_ref[...].T
    acc = pT[0:OUT_DIM, :] * hsT[0:1, :]
    for i in range(1, IN_DIM):
        acc = acc + pT[i * OUT_DIM:(i + 1) * OUT_DIM, :] * hsT[i:i + 1, :]
    o_ref[:, 0:OUT_DIM] = acc.T
    o_ref[:, OUT_DIM:OUT_DIM + 1] = jnp.ones((BLK, 1), jnp.float32)
    o_ref[:, OUT_DIM + 1:ROW] = jnp.zeros((BLK, ROW - OUT_DIM - 1), jnp.float32)


def _tc_edge(eT, h_src, w1T, b1c, w2T, b2c, *, e2):
    return pl.pallas_call(
        _tc_edge_body,
        grid=(e2 // BLK,),
        in_specs=[
            pl.BlockSpec((E_DIM, BLK), lambda i: (0, i)),
            pl.BlockSpec((BLK, IN_DIM), lambda i: (i, 0)),
            pl.BlockSpec((EDGE_H, E_DIM), lambda i: (0, 0)),
            pl.BlockSpec((EDGE_H, 1), lambda i: (0, 0)),
            pl.BlockSpec((IN_DIM * OUT_DIM, EDGE_H), lambda i: (0, 0)),
            pl.BlockSpec((IN_DIM * OUT_DIM, 1), lambda i: (0, 0)),
        ],
        out_specs=pl.BlockSpec((BLK, ROW), lambda i: (i, 0)),
        out_shape=jax.ShapeDtypeStruct((e2, ROW), jnp.float32),
    )(eT, h_src, w1T, b1c, w2T, b2c)


def _tc_final_body(parts_ref, bias_ref, gamma_ref, beta_ref, y_ref):
    p0 = parts_ref[0:N, 0:OUT_DIM]
    p1 = parts_ref[N:2 * N, 0:OUT_DIM]
    agg = p0 + p1
    deg = parts_ref[0:N, OUT_DIM:OUT_DIM + 1] \
        + parts_ref[N:2 * N, OUT_DIM:OUT_DIM + 1]
    out = agg / jnp.maximum(deg, 1.0) + bias_ref[...]
    out = jnp.maximum(out, 0.0)
    mu = jnp.mean(out, axis=0, keepdims=True)
    ctr = out - mu
    var = jnp.mean(ctr * ctr, axis=0, keepdims=True)
    y_ref[...] = gamma_ref[...] * ctr * lax.rsqrt(var + EPS) + beta_ref[...]


def _tc_final(parts, bias, gamma, beta):
    return pl.pallas_call(
        _tc_final_body,
        grid=(1,),
        in_specs=[
            pl.BlockSpec((NC * N, ROW), lambda i: (0, 0)),
            pl.BlockSpec((1, OUT_DIM), lambda i: (0, 0)),
            pl.BlockSpec((1, OUT_DIM), lambda i: (0, 0)),
            pl.BlockSpec((1, OUT_DIM), lambda i: (0, 0)),
        ],
        out_specs=pl.BlockSpec((N, OUT_DIM), lambda i: (0, 0)),
        out_shape=jax.ShapeDtypeStruct((N, OUT_DIM), jnp.float32),
    )(parts, bias.reshape(1, OUT_DIM), gamma.reshape(1, OUT_DIM),
      beta.reshape(1, OUT_DIM))


E2 = E // 2
EPW2 = E2 // NW
CH2 = EPW2 // CW
TAIL2 = EPW2 - CH2 * CW
NGRP2 = CH2 // NBUF


def _split(iw):
    i3 = iw[:, :CH2 * CW].reshape(NW, CH2, CW)
    it = iw[:, CH2 * CW:].reshape(NW, 1, TAIL2)
    return i3, it


def kernel(h, edge_index, e, W1, b1, W2, b2, bias, gamma, beta):
    src = edge_index[0].reshape(2, NW, EPW2)
    dst = edge_index[1].reshape(2, NW, EPW2)
    sc_dims = dict(ch=CH2, tail=TAIL2, ngrp=NGRP2)
    zeros = jnp.zeros((NC * N, ROW), jnp.float32)
    eT = e.T
    w1T = W1.T
    b1c = b1.reshape(EDGE_H, 1)
    w2T = W2.T.astype(jnp.bfloat16)
    b2c = b2.reshape(IN_DIM * OUT_DIM, 1)

    hsA = _sc_gather(h, *_split(src[0]), e2=E2, **sc_dims)
    hsB = _sc_gather(h, *_split(src[1]), e2=E2, **sc_dims)
    msgA = _tc_edge(eT[:, :E2], hsA, w1T, b1c, w2T, b2c, e2=E2)
    msgB = _tc_edge(eT[:, E2:], hsB, w1T, b1c, w2T, b2c, e2=E2)
    partsA = _sc_scatter(msgA, *_split(dst[0]), zeros, **sc_dims)
    partsB = _sc_scatter(msgB, *_split(dst[1]), partsA, **sc_dims)
    return _tc_final(partsB, bias, gamma, beta)


# speedup vs baseline: 1.3526x; 1.3526x over previous
import functools

import jax
import jax.numpy as jnp
from jax import lax
from jax.experimental import pallas as pl
from jax.experimental.pallas import tpu as pltpu
from jax.experimental.pallas import tpu_sc as plsc

N = 10000
E = 160000
IN_DIM = 32
OUT_DIM = 32
E_DIM = 6
EDGE_H = 256
EPS = 1e-5

ROW = 48
BLK = 3200
NB = E // BLK

NC = 2
NS = 16
NW = NC * NS
EPW = E // NW
CW = 128
CH = 39
TAIL = EPW - CH * CW
LAG = 3
NBUF = 6
NGRP = 6
NTI = 10
NPT = N // NTI


def _sc_gather(h, src3, srct, *, e2, ch, tail, ngrp):
    epw = ch * CW + tail
    mesh = plsc.VectorSubcoreMesh(core_axis_name="c", subcore_axis_name="s")

    @functools.partial(
        pl.kernel,
        out_type=jax.ShapeDtypeStruct((e2, IN_DIM), jnp.float32),
        mesh=mesh,
        scratch_types=[
            pltpu.VMEM((ch, CW), jnp.int32),
            pltpu.VMEM((1, tail), jnp.int32),
            [pltpu.VMEM((CW, IN_DIM), jnp.float32)] * NBUF,
            [pltpu.SemaphoreType.DMA] * NBUF,
            [pltpu.SemaphoreType.DMA] * NBUF,
        ],
        compiler_params=pltpu.CompilerParams(use_tc_tiling_on_sc=False),
    )
    def k(h_hbm, src_hbm, srct_hbm, out_hbm, idx_v, idxt_v, rows, gsem, wsem):
        wid = lax.axis_index("s") * NC + lax.axis_index("c")
        base = wid * epw

        def start_gather(j, b):
            pltpu.async_copy(h_hbm.at[idx_v.at[j]], rows[b], gsem[b])

        def wait_gather(j, b):
            pltpu.make_async_copy(
                h_hbm.at[idx_v.at[j]], rows[b], gsem[b]).wait()

        def start_write(j, b):
            pltpu.async_copy(
                rows[b], out_hbm.at[pl.ds(base + j * CW, CW)], wsem[b])

        def wait_write(j, b):
            pltpu.make_async_copy(
                rows[b], out_hbm.at[pl.ds(base + j * CW, CW)], wsem[b]).wait()

        pltpu.sync_copy(src_hbm.at[wid], idx_v)
        pltpu.sync_copy(srct_hbm.at[wid], idxt_v)
        for d in range(LAG):
            start_gather(d, d)

        def body(g, carry):
            for d in range(NBUF):
                j = g * NBUF + d
                bn = (d + LAG) % NBUF

                @pl.when(j >= LAG)
                def _():
                    wait_write(j - LAG, bn)

                @pl.when(j + LAG < ch)
                def _():
                    start_gather(j + LAG, bn)

                wait_gather(j, d)
                start_write(j, d)
            return carry

        lax.fori_loop(0, ngrp, body, 0)
        for j in range(ngrp * NBUF, ch):
            wait_write(j - LAG, (j - LAG) % NBUF)
            wait_gather(j, j % NBUF)
            start_write(j, j % NBUF)
        for j in range(ch - LAG, ch):
            wait_write(j, j % NBUF)
        pltpu.async_copy(h_hbm.at[idxt_v.at[0]],
                         rows[LAG].at[pl.ds(0, tail)], gsem[LAG]).wait()
        pltpu.sync_copy(rows[LAG].at[pl.ds(0, tail)],
                        out_hbm.at[pl.ds(base + ch * CW, tail)])

    return k(h, src3, srct)


def _sc_scatter(msg48, dst3, dstt, prime, *, ch, tail, ngrp):
    epw = ch * CW + tail
    mesh = plsc.VectorSubcoreMesh(core_axis_name="c", subcore_axis_name="s")

    @functools.partial(
        pl.kernel,
        out_type=jax.ShapeDtypeStruct((NC * N, ROW), jnp.float32),
        mesh=mesh,
        scratch_types=[
            pltpu.VMEM((ch, CW), jnp.int32),
            pltpu.VMEM((1, tail), jnp.int32),
            [pltpu.VMEM((CW, ROW), jnp.float32)] * NBUF,
            [pltpu.SemaphoreType.DMA] * NBUF,
            [pltpu.SemaphoreType.DMA] * NBUF,
            pltpu.SemaphoreType.DMA,
            pltpu.VMEM_SHARED((N, ROW), jnp.float32),
        ],
        compiler_params=pltpu.CompilerParams(use_tc_tiling_on_sc=False),
    )
    def k(msg_hbm, dst_hbm, dstt_hbm, zero_hbm, out_hbm, idx_v, idxt_v, vals,
          rsem, ssem, zsem, acc):
        c = lax.axis_index("c")
        s = lax.axis_index("s")
        wid = s * NC + c
        base = wid * epw

        def start_read(j, b):
            pltpu.async_copy(
                msg_hbm.at[pl.ds(base + j * CW, CW)], vals[b], rsem[b])

        def wait_read(j, b):
            pltpu.make_async_copy(
                msg_hbm.at[pl.ds(base + j * CW, CW)], vals[b], rsem[b]).wait()

        def start_scat(j, b):
            pltpu.async_copy(vals[b], acc.at[idx_v.at[j]], ssem[b], add=True)

        def wait_scat(j, b):
            pltpu.make_async_copy(
                vals[b], acc.at[idx_v.at[j]], ssem[b]).wait()

        @pl.when(s < NTI)
        def _():
            pltpu.async_copy(zero_hbm.at[pl.ds(c * N + s * NPT, NPT)],
                             acc.at[pl.ds(s * NPT, NPT)], zsem).wait()

        pltpu.sync_copy(dst_hbm.at[wid], idx_v)
        pltpu.sync_copy(dstt_hbm.at[wid], idxt_v)
        plsc.subcore_barrier()
        for d in range(LAG):
            start_read(d, d)

        def body(g, carry):
            for d in range(NBUF):
                j = g * NBUF + d
                bn = (d + LAG) % NBUF

                @pl.when(j >= LAG)
                def _():
                    wait_scat(j - LAG, bn)

                @pl.when(j + LAG < ch)
                def _():
                    start_read(j + LAG, bn)

                wait_read(j, d)
                start_scat(j, d)
            return carry

        lax.fori_loop(0, ngrp, body, 0)
        for j in range(ngrp * NBUF, ch):
            wait_scat(j - LAG, (j - LAG) % NBUF)
            wait_read(j, j % NBUF)
            start_scat(j, j % NBUF)
        for j in range(ch - LAG, ch):
            wait_scat(j, j % NBUF)
        pltpu.sync_copy(msg_hbm.at[pl.ds(base + ch * CW, tail)],
                        vals[LAG].at[pl.ds(0, tail)])
        pltpu.sync_copy(vals[LAG].at[pl.ds(0, tail)],
                        acc.at[idxt_v.at[0]], add=True)
        plsc.subcore_barrier()

        @pl.when(s < NTI)
        def _():
            pltpu.sync_copy(acc.at[pl.ds(s * NPT, NPT)],
                            out_hbm.at[pl.ds(c * N + s * NPT, NPT)])

    return k(msg48, dst3, dstt, prime)


def _tc_edge_body(eT_ref, hs_ref, w1T_ref, b1T_ref, w2T_ref, b2T_ref, o_ref):
    hidT = jnp.dot(w1T_ref[...], eT_ref[...],
                   preferred_element_type=jnp.float32)
    hidT = jnp.maximum(hidT + b1T_ref[...], 0.0).astype(jnp.bfloat16)
    pT = jnp.dot(w2T_ref[...], hidT, preferred_element_type=jnp.float32)
    pT = pT + b2T_ref[...]
    hsT = hs_ref[...].T
    acc = pT[0:OUT_DIM, :] * hsT[0:1, :]
    for i in range(1, IN_DIM):
        acc = acc + pT[i * OUT_DIM:(i + 1) * OUT_DIM, :] * hsT[i:i + 1, :]
    o_ref[:, 0:OUT_DIM] = acc.T
    o_ref[:, OUT_DIM:OUT_DIM + 1] = jnp.ones((BLK, 1), jnp.float32)
    o_ref[:, OUT_DIM + 1:ROW] = jnp.zeros((BLK, ROW - OUT_DIM - 1), jnp.float32)


def _tc_edge(eT, h_src, w1T, b1c, w2T, b2c, *, e2):
    return pl.pallas_call(
        _tc_edge_body,
        grid=(e2 // BLK,),
        in_specs=[
            pl.BlockSpec((E_DIM, BLK), lambda i: (0, i)),
            pl.BlockSpec((BLK, IN_DIM), lambda i: (i, 0)),
            pl.BlockSpec((EDGE_H, E_DIM), lambda i: (0, 0)),
            pl.BlockSpec((EDGE_H, 1), lambda i: (0, 0)),
            pl.BlockSpec((IN_DIM * OUT_DIM, EDGE_H), lambda i: (0, 0)),
            pl.BlockSpec((IN_DIM * OUT_DIM, 1), lambda i: (0, 0)),
        ],
        out_specs=pl.BlockSpec((BLK, ROW), lambda i: (i, 0)),
        out_shape=jax.ShapeDtypeStruct((e2, ROW), jnp.float32),
    )(eT, h_src, w1T, b1c, w2T, b2c)


def _tc_final_body(parts_ref, bias_ref, gamma_ref, beta_ref, y_ref):
    p0 = parts_ref[0:N, 0:OUT_DIM]
    p1 = parts_ref[N:2 * N, 0:OUT_DIM]
    agg = p0 + p1
    deg = parts_ref[0:N, OUT_DIM:OUT_DIM + 1] \
        + parts_ref[N:2 * N, OUT_DIM:OUT_DIM + 1]
    out = agg / jnp.maximum(deg, 1.0) + bias_ref[...]
    out = jnp.maximum(out, 0.0)
    mu = jnp.mean(out, axis=0, keepdims=True)
    ctr = out - mu
    var = jnp.mean(ctr * ctr, axis=0, keepdims=True)
    y_ref[...] = gamma_ref[...] * ctr * lax.rsqrt(var + EPS) + beta_ref[...]


def _tc_final(parts, bias, gamma, beta):
    return pl.pallas_call(
        _tc_final_body,
        grid=(1,),
        in_specs=[
            pl.BlockSpec((NC * N, ROW), lambda i: (0, 0)),
            pl.BlockSpec((1, OUT_DIM), lambda i: (0, 0)),
            pl.BlockSpec((1, OUT_DIM), lambda i: (0, 0)),
            pl.BlockSpec((1, OUT_DIM), lambda i: (0, 0)),
        ],
        out_specs=pl.BlockSpec((N, OUT_DIM), lambda i: (0, 0)),
        out_shape=jax.ShapeDtypeStruct((N, OUT_DIM), jnp.float32),
    )(parts, bias.reshape(1, OUT_DIM), gamma.reshape(1, OUT_DIM),
      beta.reshape(1, OUT_DIM))


E2 = E // 2
EPW2 = E2 // NW
CH2 = EPW2 // CW
TAIL2 = EPW2 - CH2 * CW
NGRP2 = CH2 // NBUF


def _split(iw):
    i3 = iw[:, :CH2 * CW].reshape(NW, CH2, CW)
    it = iw[:, CH2 * CW:].reshape(NW, 1, TAIL2)
    return i3, it


def kernel(h, edge_index, e, W1, b1, W2, b2, bias, gamma, beta):
    src = edge_index[0].reshape(2, NW, EPW2)
    dst = edge_index[1].reshape(2, NW, EPW2)
    sc_dims = dict(ch=CH2, tail=TAIL2, ngrp=NGRP2)
    zeros = jnp.zeros((NC * N, ROW), jnp.float32)
    eT = e.T
    w1T = W1.T
    b1c = b1.reshape(EDGE_H, 1)
    w2T = W2.T.astype(jnp.bfloat16)
    b2c = b2.reshape(IN_DIM * OUT_DIM, 1)

    hsA = _sc_gather(h, *_split(src[0]), e2=E2, **sc_dims)
    hsB = _sc_gather(h, *_split(src[1]), e2=E2, **sc_dims)
    msgA = _tc_edge(eT[:, :E2], hsA, w1T, b1c, w2T, b2c, e2=E2)
    msgB = _tc_edge(eT[:, E2:], hsB, w1T, b1c, w2T, b2c, e2=E2)
    partsA = _sc_scatter(msgA, *_split(dst[0]), zeros, **sc_dims)
    partsB = _sc_scatter(msgB, *_split(dst[1]), partsA, **sc_dims)
    return _tc_final(partsB, bias, gamma, beta)


# speedup vs baseline: 1.6912x; 1.2503x over previous
import functools

import jax
import jax.numpy as jnp
from jax import lax
from jax.experimental import pallas as pl
from jax.experimental.pallas import tpu as pltpu
from jax.experimental.pallas import tpu_sc as plsc

N = 10000
E = 160000
IN_DIM = 32
OUT_DIM = 32
E_DIM = 6
EDGE_H = 256
EPS = 1e-5

ROW = 128
RACC = 48
BLK = 3200
NB = E // BLK

NC = 2
NS = 16
NW = NC * NS
EPW = E // NW
CW = 128
CH = 39
TAIL = EPW - CH * CW
LAG = 3
NBUF = 6
NGRP = 6
NTI = 10
NPT = N // NTI


def _sc_gather(h, src3, srct, *, e2, ch, tail, ngrp):
    epw = ch * CW + tail
    mesh = plsc.VectorSubcoreMesh(core_axis_name="c", subcore_axis_name="s")

    @functools.partial(
        pl.kernel,
        out_type=jax.ShapeDtypeStruct((e2, IN_DIM), jnp.float32),
        mesh=mesh,
        scratch_types=[
            pltpu.VMEM((ch, CW), jnp.int32),
            pltpu.VMEM((1, tail), jnp.int32),
            [pltpu.VMEM((CW, IN_DIM), jnp.float32)] * NBUF,
            [pltpu.SemaphoreType.DMA] * NBUF,
            [pltpu.SemaphoreType.DMA] * NBUF,
        ],
        compiler_params=pltpu.CompilerParams(use_tc_tiling_on_sc=False),
    )
    def k(h_hbm, src_hbm, srct_hbm, out_hbm, idx_v, idxt_v, rows, gsem, wsem):
        wid = lax.axis_index("s") * NC + lax.axis_index("c")
        base = wid * epw

        def start_gather(j, b):
            pltpu.async_copy(h_hbm.at[idx_v.at[j]], rows[b], gsem[b])

        def wait_gather(j, b):
            pltpu.make_async_copy(
                h_hbm.at[idx_v.at[j]], rows[b], gsem[b]).wait()

        def start_write(j, b):
            pltpu.async_copy(
                rows[b], out_hbm.at[pl.ds(base + j * CW, CW)], wsem[b])

        def wait_write(j, b):
            pltpu.make_async_copy(
                rows[b], out_hbm.at[pl.ds(base + j * CW, CW)], wsem[b]).wait()

        pltpu.sync_copy(src_hbm.at[wid], idx_v)
        pltpu.sync_copy(srct_hbm.at[wid], idxt_v)
        for d in range(LAG):
            start_gather(d, d)

        def body(g, carry):
            for d in range(NBUF):
                j = g * NBUF + d
                bn = (d + LAG) % NBUF

                @pl.when(j >= LAG)
                def _():
                    wait_write(j - LAG, bn)

                @pl.when(j + LAG < ch)
                def _():
                    start_gather(j + LAG, bn)

                wait_gather(j, d)
                start_write(j, d)
            return carry

        lax.fori_loop(0, ngrp, body, 0)
        for j in range(ngrp * NBUF, ch):
            wait_write(j - LAG, (j - LAG) % NBUF)
            wait_gather(j, j % NBUF)
            start_write(j, j % NBUF)
        for j in range(ch - LAG, ch):
            wait_write(j, j % NBUF)
        pltpu.async_copy(h_hbm.at[idxt_v.at[0]],
                         rows[LAG].at[pl.ds(0, tail)], gsem[LAG]).wait()
        pltpu.sync_copy(rows[LAG].at[pl.ds(0, tail)],
                        out_hbm.at[pl.ds(base + ch * CW, tail)])

    return k(h, src3, srct)


def _sc_scatter(msg48, dst3, dstt, prime, *, ch, tail, ngrp):
    epw = ch * CW + tail
    mesh = plsc.VectorSubcoreMesh(core_axis_name="c", subcore_axis_name="s")

    @functools.partial(
        pl.kernel,
        out_type=jax.ShapeDtypeStruct((NC * N, RACC), jnp.float32),
        mesh=mesh,
        scratch_types=[
            pltpu.VMEM((ch, CW), jnp.int32),
            pltpu.VMEM((1, tail), jnp.int32),
            [pltpu.VMEM((CW, RACC), jnp.float32)] * NBUF,
            [pltpu.SemaphoreType.DMA] * NBUF,
            [pltpu.SemaphoreType.DMA] * NBUF,
            pltpu.SemaphoreType.DMA,
            pltpu.VMEM_SHARED((N, RACC), jnp.float32),
        ],
        compiler_params=pltpu.CompilerParams(use_tc_tiling_on_sc=False),
    )
    def k(msg_hbm, dst_hbm, dstt_hbm, zero_hbm, out_hbm, idx_v, idxt_v, vals,
          rsem, ssem, zsem, acc):
        c = lax.axis_index("c")
        s = lax.axis_index("s")
        wid = s * NC + c
        base = wid * epw

        def start_read(j, b):
            pltpu.async_copy(
                msg_hbm.at[pl.ds(base + j * CW, CW), pl.ds(0, RACC)],
                vals[b], rsem[b])

        def wait_read(j, b):
            pltpu.make_async_copy(
                msg_hbm.at[pl.ds(base + j * CW, CW), pl.ds(0, RACC)],
                vals[b], rsem[b]).wait()

        def start_scat(j, b):
            pltpu.async_copy(vals[b], acc.at[idx_v.at[j]], ssem[b], add=True)

        def wait_scat(j, b):
            pltpu.make_async_copy(
                vals[b], acc.at[idx_v.at[j]], ssem[b]).wait()

        @pl.when(s < NTI)
        def _():
            pltpu.async_copy(zero_hbm.at[pl.ds(c * N + s * NPT, NPT)],
                             acc.at[pl.ds(s * NPT, NPT)], zsem).wait()

        pltpu.sync_copy(dst_hbm.at[wid], idx_v)
        pltpu.sync_copy(dstt_hbm.at[wid], idxt_v)
        plsc.subcore_barrier()
        for d in range(LAG):
            start_read(d, d)

        def body(g, carry):
            for d in range(NBUF):
                j = g * NBUF + d
                bn = (d + LAG) % NBUF

                @pl.when(j >= LAG)
                def _():
                    wait_scat(j - LAG, bn)

                @pl.when(j + LAG < ch)
                def _():
                    start_read(j + LAG, bn)

                wait_read(j, d)
                start_scat(j, d)
            return carry

        lax.fori_loop(0, ngrp, body, 0)
        for j in range(ngrp * NBUF, ch):
            wait_scat(j - LAG, (j - LAG) % NBUF)
            wait_read(j, j % NBUF)
            start_scat(j, j % NBUF)
        for j in range(ch - LAG, ch):
            wait_scat(j, j % NBUF)
        pltpu.sync_copy(msg_hbm.at[pl.ds(base + ch * CW, tail), pl.ds(0, RACC)],
                        vals[LAG].at[pl.ds(0, tail)])
        pltpu.sync_copy(vals[LAG].at[pl.ds(0, tail)],
                        acc.at[idxt_v.at[0]], add=True)
        plsc.subcore_barrier()

        @pl.when(s < NTI)
        def _():
            pltpu.sync_copy(acc.at[pl.ds(s * NPT, NPT)],
                            out_hbm.at[pl.ds(c * N + s * NPT, NPT)])

    return k(msg48, dst3, dstt, prime)


def _tc_edge_body(eT_ref, hs_ref, w1T_ref, b1T_ref, w2T_ref, b2T_ref, o_ref):
    hidT = jnp.dot(w1T_ref[...], eT_ref[...],
                   preferred_element_type=jnp.float32)
    hidT = jnp.maximum(hidT + b1T_ref[...], 0.0).astype(jnp.bfloat16)
    pT = jnp.dot(w2T_ref[...], hidT, preferred_element_type=jnp.float32)
    pT = pT + b2T_ref[...]
    hsT = hs_ref[...].T
    acc = pT[0:OUT_DIM, :] * hsT[0:1, :]
    for i in range(1, IN_DIM):
        acc = acc + pT[i * OUT_DIM:(i + 1) * OUT_DIM, :] * hsT[i:i + 1, :]
    o_ref[:, 0:OUT_DIM] = acc.T
    o_ref[:, OUT_DIM:OUT_DIM + 1] = jnp.ones((BLK, 1), jnp.float32)
    o_ref[:, OUT_DIM + 1:ROW] = jnp.zeros((BLK, ROW - OUT_DIM - 1), jnp.float32)


def _tc_edge(eT, h_src, w1T, b1c, w2T, b2c, *, e2):
    return pl.pallas_call(
        _tc_edge_body,
        grid=(e2 // BLK,),
        in_specs=[
            pl.BlockSpec((E_DIM, BLK), lambda i: (0, i)),
            pl.BlockSpec((BLK, IN_DIM), lambda i: (i, 0)),
            pl.BlockSpec((EDGE_H, E_DIM), lambda i: (0, 0)),
            pl.BlockSpec((EDGE_H, 1), lambda i: (0, 0)),
            pl.BlockSpec((IN_DIM * OUT_DIM, EDGE_H), lambda i: (0, 0)),
            pl.BlockSpec((IN_DIM * OUT_DIM, 1), lambda i: (0, 0)),
        ],
        out_specs=pl.BlockSpec((BLK, ROW), lambda i: (i, 0)),
        out_shape=jax.ShapeDtypeStruct((e2, ROW), jnp.float32),
    )(eT, h_src, w1T, b1c, w2T, b2c)


def _tc_final_body(parts_ref, bias_ref, gamma_ref, beta_ref, y_ref):
    p0 = parts_ref[0:N, 0:OUT_DIM]
    p1 = parts_ref[N:2 * N, 0:OUT_DIM]
    agg = p0 + p1
    deg = parts_ref[0:N, OUT_DIM:OUT_DIM + 1] \
        + parts_ref[N:2 * N, OUT_DIM:OUT_DIM + 1]
    out = agg / jnp.maximum(deg, 1.0) + bias_ref[...]
    out = jnp.maximum(out, 0.0)
    mu = jnp.mean(out, axis=0, keepdims=True)
    ctr = out - mu
    var = jnp.mean(ctr * ctr, axis=0, keepdims=True)
    y_ref[...] = gamma_ref[...] * ctr * lax.rsqrt(var + EPS) + beta_ref[...]


def _tc_final(parts, bias, gamma, beta):
    return pl.pallas_call(
        _tc_final_body,
        grid=(1,),
        in_specs=[
            pl.BlockSpec((NC * N, RACC), lambda i: (0, 0)),
            pl.BlockSpec((1, OUT_DIM), lambda i: (0, 0)),
            pl.BlockSpec((1, OUT_DIM), lambda i: (0, 0)),
            pl.BlockSpec((1, OUT_DIM), lambda i: (0, 0)),
        ],
        out_specs=pl.BlockSpec((N, OUT_DIM), lambda i: (0, 0)),
        out_shape=jax.ShapeDtypeStruct((N, OUT_DIM), jnp.float32),
    )(parts, bias.reshape(1, OUT_DIM), gamma.reshape(1, OUT_DIM),
      beta.reshape(1, OUT_DIM))


E2 = E // 2
EPW2 = E2 // NW
CH2 = EPW2 // CW
TAIL2 = EPW2 - CH2 * CW
NGRP2 = CH2 // NBUF


def _split(iw):
    i3 = iw[:, :CH2 * CW].reshape(NW, CH2, CW)
    it = iw[:, CH2 * CW:].reshape(NW, 1, TAIL2)
    return i3, it


def kernel(h, edge_index, e, W1, b1, W2, b2, bias, gamma, beta):
    src = edge_index[0].reshape(2, NW, EPW2)
    dst = edge_index[1].reshape(2, NW, EPW2)
    sc_dims = dict(ch=CH2, tail=TAIL2, ngrp=NGRP2)
    zeros = jnp.zeros((NC * N, RACC), jnp.float32)
    eT = e.T
    w1T = W1.T
    b1c = b1.reshape(EDGE_H, 1)
    w2T = W2.T.astype(jnp.bfloat16)
    b2c = b2.reshape(IN_DIM * OUT_DIM, 1)

    hsA = _sc_gather(h, *_split(src[0]), e2=E2, **sc_dims)
    hsB = _sc_gather(h, *_split(src[1]), e2=E2, **sc_dims)
    msgA = _tc_edge(eT[:, :E2], hsA, w1T, b1c, w2T, b2c, e2=E2)
    msgB = _tc_edge(eT[:, E2:], hsB, w1T, b1c, w2T, b2c, e2=E2)
    partsA = _sc_scatter(msgA, *_split(dst[0]), zeros, **sc_dims)
    partsB = _sc_scatter(msgB, *_split(dst[1]), partsA, **sc_dims)
    return _tc_final(partsB, bias, gamma, beta)
